# Initial kernel scaffold; baseline (speedup 1.0000x reference)
import jax
import jax.numpy as jnp
from jax.experimental import pallas as pl


def kernel(x, edge_index, edge_weight, weight):
    raise NotImplementedError("write your pallas kernel here")


# speedup vs baseline: 4.9659x; 4.9659x over previous
import functools

import jax
import jax.numpy as jnp
from jax import lax
from jax.experimental import pallas as pl
from jax.experimental.pallas import tpu as pltpu
from jax.experimental.pallas import tpu_sc as plsc

N = 10000
E = 320000
D = 128

NC = 2
NS = 16
NW = NC * NS
C = 128
NCHUNK = E // C
CH_PER_W = NCHUNK // NW
CH_TAIL = NCHUNK - CH_PER_W * NW
RPT = 624
RPT_LAST = N - (NS - 1) * RPT


def _mm_body(x_ref, w_ref, o_ref):
    o_ref[...] = jnp.dot(x_ref[...], w_ref[...],
                         preferred_element_type=jnp.float32)


def _matmul(x, weight):
    return pl.pallas_call(
        _mm_body,
        grid=(5,),
        in_specs=[pl.BlockSpec((2000, D), lambda i: (i, 0)),
                  pl.BlockSpec((D, D), lambda i: (0, 0))],
        out_specs=pl.BlockSpec((2000, D), lambda i: (i, 0)),
        out_shape=jax.ShapeDtypeStruct((N, D), jnp.float32),
    )(x, weight)


def _combine_body(p_ref, o_ref):
    o_ref[...] = p_ref[0] + p_ref[1]


def _combine(partials):
    return pl.pallas_call(
        _combine_body,
        grid=(5,),
        in_specs=[pl.BlockSpec((2, 2000, D), lambda i: (0, i, 0))],
        out_specs=pl.BlockSpec((2000, D), lambda i: (i, 0)),
        out_shape=jax.ShapeDtypeStruct((N, D), jnp.float32),
    )(partials)


_mesh = plsc.VectorSubcoreMesh(core_axis_name="c", subcore_axis_name="s",
                               num_cores=NC, num_subcores=NS)


@functools.partial(
    pl.kernel,
    out_type=jax.ShapeDtypeStruct((NC, N, D), jnp.float32),
    mesh=_mesh,
    scratch_types=[
        pltpu.VMEM((C,), jnp.int32),
        pltpu.VMEM((C,), jnp.int32),
        pltpu.VMEM((C,), jnp.float32),
        pltpu.VMEM((C, D), jnp.float32),
        pltpu.VMEM_SHARED((N, D), jnp.float32),
        pltpu.SemaphoreType.DMA,
    ],
)
def _spmm_sc(support_hbm, src_hbm, dst_hbm, wts_hbm, zeros_hbm, out_hbm,
             src_v, dst_v, wts_v, rows_v, acc, sem):
    c = lax.axis_index("c")
    s = lax.axis_index("s")
    wid = s * NC + c

    @pl.when(s < NS - 1)
    def _zero_main():
        pltpu.sync_copy(zeros_hbm.at[pl.ds(0, RPT)],
                        acc.at[pl.ds(s * RPT, RPT)])

    @pl.when(s == NS - 1)
    def _zero_last():
        pltpu.sync_copy(zeros_hbm, acc.at[pl.ds((NS - 1) * RPT, RPT_LAST)])

    plsc.subcore_barrier()

    def do_chunk(cid):
        base = cid * C
        pltpu.sync_copy(src_hbm.at[pl.ds(base, C)], src_v)
        pltpu.sync_copy(dst_hbm.at[pl.ds(base, C)], dst_v)
        pltpu.sync_copy(wts_hbm.at[pl.ds(base, C)], wts_v)
        pltpu.async_copy(support_hbm.at[src_v], rows_v, sem).wait()

        def scale_group(g, carry):
            wvec = wts_v[pl.ds(g * 16, 16)]
            for el in range(16):
                wsp = wvec[jnp.full((16,), el, jnp.int32)]
                e = g * 16 + el
                for j in range(D // 16):
                    sl = pl.ds(j * 16, 16)
                    rows_v[e, sl] = rows_v[e, sl] * wsp
            return carry

        lax.fori_loop(0, C // 16, scale_group, 0)
        pltpu.sync_copy(rows_v, acc.at[dst_v], add=True)

    def body(t, carry):
        do_chunk(wid + t * NW)
        return carry

    lax.fori_loop(0, CH_PER_W, body, 0)

    @pl.when(wid < CH_TAIL)
    def _tail():
        do_chunk(CH_PER_W * NW + wid)

    plsc.subcore_barrier()

    @pl.when(s < NS - 1)
    def _out_main():
        r0 = s * RPT
        pltpu.sync_copy(acc.at[pl.ds(r0, RPT)],
                        out_hbm.at[c, pl.ds(r0, RPT)])

    @pl.when(s == NS - 1)
    def _out_last():
        r0 = (NS - 1) * RPT
        pltpu.sync_copy(acc.at[pl.ds(r0, RPT_LAST)],
                        out_hbm.at[c, pl.ds(r0, RPT_LAST)])


def kernel(x, edge_index, edge_weight, weight):
    support = _matmul(x, weight)
    dst = edge_index[0]
    src = edge_index[1]
    zeros = jnp.zeros((RPT_LAST, D), jnp.float32)
    partials = _spmm_sc(support, src, dst, edge_weight, zeros)
    return _combine(partials)
